# baseline (speedup 1.0000x reference)
import functools

import jax
import jax.numpy as jnp
from jax import lax
from jax.experimental import pallas as pl
from jax.experimental.pallas import tpu as pltpu
from jax.experimental.pallas import tpu_sc as plsc

N = 10000
E = 160000
D_IN = 128
D_H = 128
C = 16

NC = 2
NS = 16
NW = NC * NS
L = 16

NPAD = 10240
STRIPE = NPAD // NS
K = 128
NCH = 40
EPAD = NW * NCH * K
EW_PER = NCH * K

_mesh = plsc.VectorSubcoreMesh(core_axis_name="c", subcore_axis_name="s")


def _make_agg(D):
    @functools.partial(
        pl.kernel,
        mesh=_mesh,
        out_type=jax.ShapeDtypeStruct((NC, NPAD, D), jnp.float32),
        compiler_params=pltpu.CompilerParams(use_tc_tiling_on_sc=(D >= 128)),
        scratch_types=[
            pltpu.VMEM((NCH, K), jnp.int32),
            pltpu.VMEM((NCH, K), jnp.int32),
            pltpu.VMEM((K, D), jnp.float32),
            pltpu.VMEM((K, D), jnp.float32),
            pltpu.SemaphoreType.DMA,
            pltpu.SemaphoreType.DMA,
            pltpu.VMEM_SHARED((NPAD, D), jnp.float32),
        ],
    )
    def agg(table, gidx, sidx, zrows, out,
            gidx_v, sidx_v, rows_a, rows_b, sem_a, sem_b, acc):
        c = lax.axis_index("c")
        s = lax.axis_index("s")
        w = c * NS + s
        pltpu.sync_copy(gidx.at[w], gidx_v)
        pltpu.sync_copy(sidx.at[w], sidx_v)
        pltpu.sync_copy(zrows, acc.at[pl.ds(s * STRIPE, STRIPE)])
        plsc.subcore_barrier()

        pltpu.async_copy(table.at[gidx_v.at[0]], rows_a, sem_a)

        def body(j2, carry):
            ja = 2 * j2
            pltpu.async_copy(table.at[gidx_v.at[ja + 1]], rows_b, sem_b)
            pltpu.make_async_copy(table.at[gidx_v.at[ja]], rows_a, sem_a).wait()
            pltpu.sync_copy(rows_a, acc.at[sidx_v.at[ja]], add=True)

            @pl.when(j2 < NCH // 2 - 1)
            def _():
                pltpu.async_copy(table.at[gidx_v.at[ja + 2]], rows_a, sem_a)

            pltpu.make_async_copy(
                table.at[gidx_v.at[ja + 1]], rows_b, sem_b).wait()
            pltpu.sync_copy(rows_b, acc.at[sidx_v.at[ja + 1]], add=True)
            return carry

        lax.fori_loop(0, NCH // 2, body, 0)
        plsc.subcore_barrier()
        pltpu.sync_copy(acc.at[pl.ds(s * STRIPE, STRIPE)],
                        out.at[c].at[pl.ds(s * STRIPE, STRIPE)])

    return agg


_agg128 = _make_agg(D_H)


@functools.partial(
    pl.kernel,
    mesh=_mesh,
    out_type=(jax.ShapeDtypeStruct((NC, NPAD, C), jnp.float32),
              jax.ShapeDtypeStruct((NC, NPAD, C), jnp.float32)),
    compiler_params=pltpu.CompilerParams(use_tc_tiling_on_sc=False),
    scratch_types=[
        pltpu.VMEM((NCH, K), jnp.int32),
        pltpu.VMEM((NCH, K), jnp.int32),
        pltpu.VMEM((NCH, K), jnp.int32),
        pltpu.VMEM((K, C), jnp.float32),
        pltpu.VMEM((K, C), jnp.float32),
        pltpu.VMEM((K, C), jnp.float32),
        pltpu.SemaphoreType.DMA,
        pltpu.SemaphoreType.DMA,
        pltpu.VMEM_SHARED((NPAD, C), jnp.float32),
        pltpu.VMEM_SHARED((NPAD, C), jnp.float32),
    ],
)
def _cnt_lpa_kernel(l0, gidx, sidxq, sidxc, const_rows, zrows, cnt_out, q_out,
                    g_v, sq_v, sc_v, rows_a, rows_b, const_v, sem_a, sem_b,
                    acc_c, acc_q):
    c = lax.axis_index("c")
    s = lax.axis_index("s")
    w = c * NS + s
    pltpu.sync_copy(gidx.at[w], g_v)
    pltpu.sync_copy(sidxq.at[w], sq_v)
    pltpu.sync_copy(sidxc.at[w], sc_v)
    pltpu.sync_copy(const_rows, const_v)
    pltpu.sync_copy(zrows, acc_c.at[pl.ds(s * STRIPE, STRIPE)])
    pltpu.sync_copy(zrows, acc_q.at[pl.ds(s * STRIPE, STRIPE)])
    plsc.subcore_barrier()

    pltpu.async_copy(l0.at[g_v.at[0]], rows_a, sem_a)

    def body(j2, carry):
        ja = 2 * j2
        pltpu.async_copy(l0.at[g_v.at[ja + 1]], rows_b, sem_b)
        pltpu.make_async_copy(l0.at[g_v.at[ja]], rows_a, sem_a).wait()
        pltpu.sync_copy(rows_a, acc_q.at[sq_v.at[ja]], add=True)
        pltpu.sync_copy(const_v, acc_c.at[sc_v.at[ja]], add=True)

        @pl.when(j2 < NCH // 2 - 1)
        def _():
            pltpu.async_copy(l0.at[g_v.at[ja + 2]], rows_a, sem_a)

        pltpu.make_async_copy(l0.at[g_v.at[ja + 1]], rows_b, sem_b).wait()
        pltpu.sync_copy(rows_b, acc_q.at[sq_v.at[ja + 1]], add=True)
        pltpu.sync_copy(const_v, acc_c.at[sc_v.at[ja + 1]], add=True)
        return carry

    lax.fori_loop(0, NCH // 2, body, 0)
    plsc.subcore_barrier()
    pltpu.sync_copy(acc_c.at[pl.ds(s * STRIPE, STRIPE)],
                    cnt_out.at[c].at[pl.ds(s * STRIPE, STRIPE)])
    pltpu.sync_copy(acc_q.at[pl.ds(s * STRIPE, STRIPE)],
                    q_out.at[c].at[pl.ds(s * STRIPE, STRIPE)])


@functools.partial(
    pl.kernel,
    mesh=_mesh,
    out_type=(jax.ShapeDtypeStruct((NC, NPAD, C), jnp.float32),
              jax.ShapeDtypeStruct((NC, NPAD, C), jnp.float32)),
    compiler_params=pltpu.CompilerParams(use_tc_tiling_on_sc=False),
    scratch_types=[
        pltpu.VMEM((NCH, K), jnp.int32),
        pltpu.VMEM((NCH, K), jnp.int32),
        pltpu.VMEM((NCH, K), jnp.int32),
        pltpu.VMEM((NCH, K), jnp.int32),
        pltpu.VMEM((K, C), jnp.float32),
        pltpu.VMEM((K, C), jnp.float32),
        pltpu.VMEM((K, C), jnp.float32),
        pltpu.VMEM((K, C), jnp.float32),
        pltpu.SemaphoreType.DMA,
        pltpu.SemaphoreType.DMA,
        pltpu.SemaphoreType.DMA,
        pltpu.SemaphoreType.DMA,
        pltpu.VMEM_SHARED((NPAD, C), jnp.float32),
        pltpu.VMEM_SHARED((NPAD, C), jnp.float32),
    ],
)
def _dualc_kernel(t1, g1i, s1i, t2, g2i, s2i, zrows, o1, o2,
                  g1_v, s1_v, g2_v, s2_v, ra1, rb1, ra2, rb2,
                  sa1, sb1, sa2, sb2, acc1, acc2):
    c = lax.axis_index("c")
    s = lax.axis_index("s")
    w = c * NS + s
    pltpu.sync_copy(g1i.at[w], g1_v)
    pltpu.sync_copy(s1i.at[w], s1_v)
    pltpu.sync_copy(g2i.at[w], g2_v)
    pltpu.sync_copy(s2i.at[w], s2_v)
    pltpu.sync_copy(zrows, acc1.at[pl.ds(s * STRIPE, STRIPE)])
    pltpu.sync_copy(zrows, acc2.at[pl.ds(s * STRIPE, STRIPE)])
    plsc.subcore_barrier()

    pltpu.async_copy(t1.at[g1_v.at[0]], ra1, sa1)
    pltpu.async_copy(t2.at[g2_v.at[0]], ra2, sa2)

    def body(j2, carry):
        ja = 2 * j2
        pltpu.async_copy(t1.at[g1_v.at[ja + 1]], rb1, sb1)
        pltpu.async_copy(t2.at[g2_v.at[ja + 1]], rb2, sb2)
        pltpu.make_async_copy(t1.at[g1_v.at[ja]], ra1, sa1).wait()
        pltpu.sync_copy(ra1, acc1.at[s1_v.at[ja]], add=True)
        pltpu.make_async_copy(t2.at[g2_v.at[ja]], ra2, sa2).wait()
        pltpu.sync_copy(ra2, acc2.at[s2_v.at[ja]], add=True)

        @pl.when(j2 < NCH // 2 - 1)
        def _():
            pltpu.async_copy(t1.at[g1_v.at[ja + 2]], ra1, sa1)
            pltpu.async_copy(t2.at[g2_v.at[ja + 2]], ra2, sa2)

        pltpu.make_async_copy(t1.at[g1_v.at[ja + 1]], rb1, sb1).wait()
        pltpu.sync_copy(rb1, acc1.at[s1_v.at[ja + 1]], add=True)
        pltpu.make_async_copy(t2.at[g2_v.at[ja + 1]], rb2, sb2).wait()
        pltpu.sync_copy(rb2, acc2.at[s2_v.at[ja + 1]], add=True)
        return carry

    lax.fori_loop(0, NCH // 2, body, 0)
    plsc.subcore_barrier()
    pltpu.sync_copy(acc1.at[pl.ds(s * STRIPE, STRIPE)],
                    o1.at[c].at[pl.ds(s * STRIPE, STRIPE)])
    pltpu.sync_copy(acc2.at[pl.ds(s * STRIPE, STRIPE)],
                    o2.at[c].at[pl.ds(s * STRIPE, STRIPE)])


def _sig1():
    return 1.0 / (1.0 + jnp.exp(jnp.float32(-1.0)))


def _onehot_body(yrep, l0_o):
    cidx = lax.broadcasted_iota(jnp.int32, (1, 128), 1) % C
    l0_o[...] = (yrep[...] == cidx).astype(jnp.float32)


def _prep_body(degp, x, w1, dinv_o, t1_o, g1_o):
    deg = _sig1() * (degp[0, :N, 0] + degp[1, :N, 0]) + 1.0
    dinv = lax.rsqrt(deg)[:, None]
    dinv_o[...] = dinv
    t1 = jnp.dot(x[...], w1[...], preferred_element_type=jnp.float32)
    t1_o[...] = t1
    g1_o[...] = (_sig1() * dinv) * t1


def _layer_body(p, t, dinv_r, b, gm, bt, wn, tn_o, gn_o):
    dinv = dinv_r[...]
    z = dinv * (p[0, :N, :] + p[1, :N, :]) + (dinv * dinv) * t[...] + b[...]
    mu = jnp.mean(z, axis=0, keepdims=True)
    var = jnp.mean((z - mu) ** 2, axis=0, keepdims=True)
    h = jnp.maximum((z - mu) * lax.rsqrt(var + 1e-5) * gm[...] + bt[...], 0.0)
    tn = jnp.dot(h, wn[...], preferred_element_type=jnp.float32)
    tn_o[...] = tn
    gn_o[...] = (_sig1() * dinv) * tn


def _lpa_body(q, lprev, l_o):
    l_o[...] = _sig1() * (q[0, :N // 8, :] + q[1, :N // 8, :]) + lprev[...]


def _final_body(p, t, dinv_r, b, out_o):
    dinv = dinv_r[...]
    z = dinv * (p[0, :N, :] + p[1, :N, :]) + (dinv * dinv) * t[...] + b[...]
    m = jnp.max(z, axis=1, keepdims=True)
    e = jnp.exp(z - m)
    out_o[...] = e / jnp.sum(e, axis=1, keepdims=True)


def _lpa_final_body(q, lprev, l_o):
    l2 = _sig1() * (q[0, :N, :] + q[1, :N, :]) + lprev[...]
    nrm = jnp.sqrt(jnp.sum(l2 * l2, axis=1, keepdims=True))
    l_o[...] = l2 / jnp.maximum(nrm, 1e-12)


def _tc(body, out_shape, *args):
    return pl.pallas_call(body, out_shape=out_shape)(*args)


_f32 = jnp.float32


def kernel(x, edge_index, y, edge_weight, W1, b1, W2, b2, W3, b3,
           g1, bt1, g2, bt2):
    padn = EPAD - E
    NR = N // 8
    NPR = NPAD // 8
    ER = E // K
    PR = padn // K
    ei3 = edge_index.reshape(2, ER, K)
    pidx2 = jnp.arange(padn, dtype=jnp.int32).reshape(PR, K)
    pad_g2 = pidx2 % N
    pad_s2 = N + pidx2 % (NPAD - N)
    gidx_conv = jnp.concatenate([ei3[0], pad_g2], 0).reshape(NW, NCH, K)
    sidx_conv = jnp.concatenate([ei3[1], pad_s2], 0).reshape(NW, NCH, K)
    gidx_lpa = jnp.concatenate([ei3[1], pad_g2], 0).reshape(NW, NCH, K)
    sidx_lpa = jnp.concatenate([ei3[0], pad_s2], 0).reshape(NW, NCH, K)
    const_rows = jnp.tile(
        (jnp.arange(C, dtype=jnp.int32) == 0).astype(_f32)[None, :], (K, 1))
    zrows128 = jnp.zeros((STRIPE, D_H), _f32)
    zrowsC = jnp.zeros((STRIPE, C), _f32)
    yrep = jnp.repeat(y.astype(jnp.int32).reshape(NR, 8), C, axis=1)
    b1r, b2r = b1.reshape(1, D_H), b2.reshape(1, D_H)
    b3r = b3.reshape(1, C)
    g1r, bt1r = g1.reshape(1, D_H), bt1.reshape(1, D_H)
    g2r, bt2r = g2.reshape(1, D_H), bt2.reshape(1, D_H)

    L0_128 = _tc(_onehot_body, jax.ShapeDtypeStruct((NR, 128), _f32), yrep)
    L0 = L0_128.reshape(N, C)
    degp, q1 = _cnt_lpa_kernel(L0, gidx_lpa, sidx_lpa, sidx_conv,
                               const_rows, zrowsC)
    dinv, t1, g1t = _tc(
        _prep_body,
        (jax.ShapeDtypeStruct((N, 1), _f32),
         jax.ShapeDtypeStruct((N, D_H), _f32),
         jax.ShapeDtypeStruct((N, D_H), _f32)),
        degp, x, W1)

    p1 = _agg128(g1t, gidx_conv, sidx_conv, zrows128)
    t2, g2t = _tc(
        _layer_body,
        (jax.ShapeDtypeStruct((N, D_H), _f32),
         jax.ShapeDtypeStruct((N, D_H), _f32)),
        p1, t1, dinv, b1r, g1r, bt1r, W2)

    p2 = _agg128(g2t, gidx_conv, sidx_conv, zrows128)
    t3, g3t = _tc(
        _layer_body,
        (jax.ShapeDtypeStruct((N, C), _f32),
         jax.ShapeDtypeStruct((N, C), _f32)),
        p2, t2, dinv, b2r, g2r, bt2r, W3)
    L1_128 = _tc(_lpa_body, jax.ShapeDtypeStruct((NR, 128), _f32),
                 q1.reshape(NC, NPR, 128), L0_128)
    L1 = L1_128.reshape(N, C)

    p3, q2 = _dualc_kernel(g3t, gidx_conv, sidx_conv,
                           L1, gidx_lpa, sidx_lpa, zrowsC)
    out1 = _tc(_final_body, jax.ShapeDtypeStruct((N, C), _f32),
               p3, t3, dinv, b3r)
    labels = _tc(_lpa_final_body, jax.ShapeDtypeStruct((N, C), _f32), q2, L1)

    return (out1, labels)


# speedup vs baseline: 1.0190x; 1.0190x over previous
import functools

import jax
import jax.numpy as jnp
from jax import lax
from jax.experimental import pallas as pl
from jax.experimental.pallas import tpu as pltpu
from jax.experimental.pallas import tpu_sc as plsc

N = 10000
E = 160000
D_IN = 128
D_H = 128
C = 16

NC = 2
NS = 16
NW = NC * NS
L = 16

NPAD = 10240
STRIPE = NPAD // NS
K = 128
NCH = 40
EPAD = NW * NCH * K
EW_PER = NCH * K

_mesh = plsc.VectorSubcoreMesh(core_axis_name="c", subcore_axis_name="s")


def _make_agg(D):
    @functools.partial(
        pl.kernel,
        mesh=_mesh,
        out_type=jax.ShapeDtypeStruct((NC, NPAD, D), jnp.float32),
        compiler_params=pltpu.CompilerParams(use_tc_tiling_on_sc=(D >= 128)),
        scratch_types=[
            pltpu.VMEM((NCH, K), jnp.int32),
            pltpu.VMEM((NCH, K), jnp.int32),
            pltpu.VMEM((K, D), jnp.float32),
            pltpu.VMEM((K, D), jnp.float32),
            pltpu.SemaphoreType.DMA,
            pltpu.SemaphoreType.DMA,
            pltpu.VMEM_SHARED((NPAD, D), jnp.float32),
        ],
    )
    def agg(table, gidx, sidx, zrows, out,
            gidx_v, sidx_v, rows_a, rows_b, sem_a, sem_b, acc):
        c = lax.axis_index("c")
        s = lax.axis_index("s")
        w = c * NS + s
        pltpu.sync_copy(gidx.at[w], gidx_v)
        pltpu.sync_copy(sidx.at[w], sidx_v)
        pltpu.sync_copy(zrows, acc.at[pl.ds(s * STRIPE, STRIPE)])
        plsc.subcore_barrier()

        pltpu.async_copy(table.at[gidx_v.at[0]], rows_a, sem_a)

        def body(j2, carry):
            ja = 2 * j2
            pltpu.async_copy(table.at[gidx_v.at[ja + 1]], rows_b, sem_b)
            pltpu.make_async_copy(table.at[gidx_v.at[ja]], rows_a, sem_a).wait()
            pltpu.sync_copy(rows_a, acc.at[sidx_v.at[ja]], add=True)

            @pl.when(j2 < NCH // 2 - 1)
            def _():
                pltpu.async_copy(table.at[gidx_v.at[ja + 2]], rows_a, sem_a)

            pltpu.make_async_copy(
                table.at[gidx_v.at[ja + 1]], rows_b, sem_b).wait()
            pltpu.sync_copy(rows_b, acc.at[sidx_v.at[ja + 1]], add=True)
            return carry

        lax.fori_loop(0, NCH // 2, body, 0)
        plsc.subcore_barrier()
        pltpu.sync_copy(acc.at[pl.ds(s * STRIPE, STRIPE)],
                        out.at[c].at[pl.ds(s * STRIPE, STRIPE)])

    return agg


_agg128 = _make_agg(D_H)
_aggC = _make_agg(C)


@functools.partial(
    pl.kernel,
    mesh=_mesh,
    out_type=(jax.ShapeDtypeStruct((NC, NPAD, C), jnp.float32),
              jax.ShapeDtypeStruct((NC, NPAD, C), jnp.float32)),
    compiler_params=pltpu.CompilerParams(use_tc_tiling_on_sc=False),
    scratch_types=[
        pltpu.VMEM((NCH, K), jnp.int32),
        pltpu.VMEM((NCH, K), jnp.int32),
        pltpu.VMEM((NCH, K), jnp.int32),
        pltpu.VMEM((K, C), jnp.float32),
        pltpu.VMEM((K, C), jnp.float32),
        pltpu.VMEM((K, C), jnp.float32),
        pltpu.SemaphoreType.DMA,
        pltpu.SemaphoreType.DMA,
        pltpu.VMEM_SHARED((NPAD, C), jnp.float32),
        pltpu.VMEM_SHARED((NPAD, C), jnp.float32),
    ],
)
def _cnt_lpa_kernel(l0, gidx, sidxq, sidxc, const_rows, zrows, cnt_out, q_out,
                    g_v, sq_v, sc_v, rows_a, rows_b, const_v, sem_a, sem_b,
                    acc_c, acc_q):
    c = lax.axis_index("c")
    s = lax.axis_index("s")
    w = c * NS + s
    pltpu.sync_copy(gidx.at[w], g_v)
    pltpu.sync_copy(sidxq.at[w], sq_v)
    pltpu.sync_copy(sidxc.at[w], sc_v)
    pltpu.sync_copy(const_rows, const_v)
    pltpu.sync_copy(zrows, acc_c.at[pl.ds(s * STRIPE, STRIPE)])
    pltpu.sync_copy(zrows, acc_q.at[pl.ds(s * STRIPE, STRIPE)])
    plsc.subcore_barrier()

    pltpu.async_copy(l0.at[g_v.at[0]], rows_a, sem_a)

    def body(j2, carry):
        ja = 2 * j2
        pltpu.async_copy(l0.at[g_v.at[ja + 1]], rows_b, sem_b)
        pltpu.make_async_copy(l0.at[g_v.at[ja]], rows_a, sem_a).wait()
        pltpu.sync_copy(rows_a, acc_q.at[sq_v.at[ja]], add=True)
        pltpu.sync_copy(const_v, acc_c.at[sc_v.at[ja]], add=True)

        @pl.when(j2 < NCH // 2 - 1)
        def _():
            pltpu.async_copy(l0.at[g_v.at[ja + 2]], rows_a, sem_a)

        pltpu.make_async_copy(l0.at[g_v.at[ja + 1]], rows_b, sem_b).wait()
        pltpu.sync_copy(rows_b, acc_q.at[sq_v.at[ja + 1]], add=True)
        pltpu.sync_copy(const_v, acc_c.at[sc_v.at[ja + 1]], add=True)
        return carry

    lax.fori_loop(0, NCH // 2, body, 0)
    plsc.subcore_barrier()
    pltpu.sync_copy(acc_c.at[pl.ds(s * STRIPE, STRIPE)],
                    cnt_out.at[c].at[pl.ds(s * STRIPE, STRIPE)])
    pltpu.sync_copy(acc_q.at[pl.ds(s * STRIPE, STRIPE)],
                    q_out.at[c].at[pl.ds(s * STRIPE, STRIPE)])


def _sig1():
    return 1.0 / (1.0 + jnp.exp(jnp.float32(-1.0)))


def _onehot_body(yrep, l0_o):
    cidx = lax.broadcasted_iota(jnp.int32, (1, 128), 1) % C
    l0_o[...] = (yrep[...] == cidx).astype(jnp.float32)


def _mm1_body(x, w1, t1_o):
    t1_o[...] = jnp.dot(x[...], w1[...], preferred_element_type=jnp.float32)


def _scale_body(degp, t1, dinv_o, g1_o):
    deg = _sig1() * (degp[0, :N, 0] + degp[1, :N, 0]) + 1.0
    dinv = lax.rsqrt(deg)[:, None]
    dinv_o[...] = dinv
    g1_o[...] = (_sig1() * dinv) * t1[...]


def _layer_body(p, t, dinv_r, b, gm, bt, wn, tn_o, gn_o):
    dinv = dinv_r[...]
    z = dinv * (p[0, :N, :] + p[1, :N, :]) + (dinv * dinv) * t[...] + b[...]
    mu = jnp.mean(z, axis=0, keepdims=True)
    var = jnp.mean((z - mu) ** 2, axis=0, keepdims=True)
    h = jnp.maximum((z - mu) * lax.rsqrt(var + 1e-5) * gm[...] + bt[...], 0.0)
    tn = jnp.dot(h, wn[...], preferred_element_type=jnp.float32)
    tn_o[...] = tn
    gn_o[...] = (_sig1() * dinv) * tn


def _lpa_body(q, lprev, l_o):
    l_o[...] = _sig1() * (q[0, :N // 8, :] + q[1, :N // 8, :]) + lprev[...]


def _final_body(p, t, dinv_r, b, out_o):
    dinv = dinv_r[...]
    z = dinv * (p[0, :N, :] + p[1, :N, :]) + (dinv * dinv) * t[...] + b[...]
    m = jnp.max(z, axis=1, keepdims=True)
    e = jnp.exp(z - m)
    out_o[...] = e / jnp.sum(e, axis=1, keepdims=True)


def _lpa_final_body(q, lprev, l_o):
    l2 = _sig1() * (q[0, :N, :] + q[1, :N, :]) + lprev[...]
    nrm = jnp.sqrt(jnp.sum(l2 * l2, axis=1, keepdims=True))
    l_o[...] = l2 / jnp.maximum(nrm, 1e-12)


def _tc(body, out_shape, *args):
    return pl.pallas_call(body, out_shape=out_shape)(*args)


_f32 = jnp.float32


def kernel(x, edge_index, y, edge_weight, W1, b1, W2, b2, W3, b3,
           g1, bt1, g2, bt2):
    padn = EPAD - E
    NR = N // 8
    NPR = NPAD // 8
    ER = E // K
    PR = padn // K
    ei3 = edge_index.reshape(2, ER, K)
    pidx2 = jnp.arange(padn, dtype=jnp.int32).reshape(PR, K)
    pad_g2 = pidx2 % N
    pad_s2 = N + pidx2 % (NPAD - N)
    gidx_conv = jnp.concatenate([ei3[0], pad_g2], 0).reshape(NW, NCH, K)
    sidx_conv = jnp.concatenate([ei3[1], pad_s2], 0).reshape(NW, NCH, K)
    gidx_lpa = jnp.concatenate([ei3[1], pad_g2], 0).reshape(NW, NCH, K)
    sidx_lpa = jnp.concatenate([ei3[0], pad_s2], 0).reshape(NW, NCH, K)
    const_rows = jnp.tile(
        (jnp.arange(C, dtype=jnp.int32) == 0).astype(_f32)[None, :], (K, 1))
    zrows128 = jnp.zeros((STRIPE, D_H), _f32)
    zrowsC = jnp.zeros((STRIPE, C), _f32)
    yrep = jnp.repeat(y.astype(jnp.int32).reshape(NR, 8), C, axis=1)
    b1r, b2r = b1.reshape(1, D_H), b2.reshape(1, D_H)
    b3r = b3.reshape(1, C)
    g1r, bt1r = g1.reshape(1, D_H), bt1.reshape(1, D_H)
    g2r, bt2r = g2.reshape(1, D_H), bt2.reshape(1, D_H)

    L0_128 = _tc(_onehot_body, jax.ShapeDtypeStruct((NR, 128), _f32), yrep)
    L0 = L0_128.reshape(N, C)
    t1 = _tc(_mm1_body, jax.ShapeDtypeStruct((N, D_H), _f32), x, W1)
    degp, q1 = _cnt_lpa_kernel(L0, gidx_lpa, sidx_lpa, sidx_conv,
                               const_rows, zrowsC)
    dinv, g1t = _tc(
        _scale_body,
        (jax.ShapeDtypeStruct((N, 1), _f32),
         jax.ShapeDtypeStruct((N, D_H), _f32)),
        degp, t1)

    p1 = _agg128(g1t, gidx_conv, sidx_conv, zrows128)
    t2, g2t = _tc(
        _layer_body,
        (jax.ShapeDtypeStruct((N, D_H), _f32),
         jax.ShapeDtypeStruct((N, D_H), _f32)),
        p1, t1, dinv, b1r, g1r, bt1r, W2)

    p2 = _agg128(g2t, gidx_conv, sidx_conv, zrows128)
    t3, g3t = _tc(
        _layer_body,
        (jax.ShapeDtypeStruct((N, C), _f32),
         jax.ShapeDtypeStruct((N, C), _f32)),
        p2, t2, dinv, b2r, g2r, bt2r, W3)
    L1_128 = _tc(_lpa_body, jax.ShapeDtypeStruct((NR, 128), _f32),
                 q1.reshape(NC, NPR, 128), L0_128)
    L1 = L1_128.reshape(N, C)

    p3 = _aggC(g3t, gidx_conv, sidx_conv, zrowsC)
    q2 = _aggC(L1, gidx_lpa, sidx_lpa, zrowsC)
    out1 = _tc(_final_body, jax.ShapeDtypeStruct((N, C), _f32),
               p3, t3, dinv, b3r)
    labels = _tc(_lpa_final_body, jax.ShapeDtypeStruct((N, C), _f32), q2, L1)

    return (out1, labels)


# speedup vs baseline: 1.0951x; 1.0747x over previous
import functools

import jax
import jax.numpy as jnp
from jax import lax
from jax.experimental import pallas as pl
from jax.experimental.pallas import tpu as pltpu
from jax.experimental.pallas import tpu_sc as plsc

N = 10000
E = 160000
D_IN = 128
D_H = 128
C = 16

NC = 2
NS = 16
NW = NC * NS
L = 16

NPAD = 10240
STRIPE = NPAD // NS
K = 128
NCH = 40
EPAD = NW * NCH * K
EW_PER = NCH * K

_mesh = plsc.VectorSubcoreMesh(core_axis_name="c", subcore_axis_name="s")


def _make_agg(D, NB):
    @functools.partial(
        pl.kernel,
        mesh=_mesh,
        out_type=jax.ShapeDtypeStruct((NC, NPAD, D), jnp.float32),
        compiler_params=pltpu.CompilerParams(use_tc_tiling_on_sc=(D >= 128)),
        scratch_types=(
            [pltpu.VMEM((NCH, K), jnp.int32),
             pltpu.VMEM((NCH, K), jnp.int32)]
            + [pltpu.VMEM((K, D), jnp.float32)] * NB
            + [pltpu.SemaphoreType.DMA] * NB
            + [pltpu.VMEM_SHARED((NPAD, D), jnp.float32)]
        ),
    )
    def agg(table, gidx, sidx, zrows, out, *scr):
        gidx_v, sidx_v = scr[0], scr[1]
        rows = scr[2:2 + NB]
        sems = scr[2 + NB:2 + 2 * NB]
        acc = scr[2 + 2 * NB]
        c = lax.axis_index("c")
        s = lax.axis_index("s")
        w = c * NS + s
        pltpu.sync_copy(gidx.at[w], gidx_v)
        pltpu.sync_copy(sidx.at[w], sidx_v)
        pltpu.sync_copy(zrows, acc.at[pl.ds(s * STRIPE, STRIPE)])
        plsc.subcore_barrier()

        for b in range(NB - 1):
            pltpu.async_copy(table.at[gidx_v.at[b]], rows[b], sems[b])

        def body(j, carry):
            ja = NB * j
            pltpu.async_copy(
                table.at[gidx_v.at[ja + NB - 1]], rows[NB - 1], sems[NB - 1])
            for b in range(NB):
                pltpu.make_async_copy(
                    table.at[gidx_v.at[ja + b]], rows[b], sems[b]).wait()
                pltpu.sync_copy(rows[b], acc.at[sidx_v.at[ja + b]], add=True)
                if b < NB - 1:
                    @pl.when(ja + NB + b < NCH)
                    def _():
                        pltpu.async_copy(
                            table.at[gidx_v.at[ja + NB + b]], rows[b], sems[b])
            return carry

        lax.fori_loop(0, NCH // NB, body, 0)
        plsc.subcore_barrier()
        pltpu.sync_copy(acc.at[pl.ds(s * STRIPE, STRIPE)],
                        out.at[c].at[pl.ds(s * STRIPE, STRIPE)])

    return agg


_agg128 = _make_agg(D_H, 2)
_aggC = _make_agg(C, 4)


@functools.partial(
    pl.kernel,
    mesh=_mesh,
    out_type=(jax.ShapeDtypeStruct((NC, NPAD, C), jnp.float32),
              jax.ShapeDtypeStruct((NC, NPAD, C), jnp.float32)),
    compiler_params=pltpu.CompilerParams(use_tc_tiling_on_sc=False),
    scratch_types=(
        [pltpu.VMEM((NCH, K), jnp.int32),
         pltpu.VMEM((NCH, K), jnp.int32),
         pltpu.VMEM((NCH, K), jnp.int32)]
        + [pltpu.VMEM((K, C), jnp.float32)] * 4
        + [pltpu.VMEM((K, C), jnp.float32)]
        + [pltpu.SemaphoreType.DMA] * 4
        + [pltpu.VMEM_SHARED((NPAD, C), jnp.float32),
           pltpu.VMEM_SHARED((NPAD, C), jnp.float32)]
    ),
)
def _cnt_lpa_kernel(l0, gidx, sidxq, sidxc, const_rows, zrows, cnt_out, q_out,
                    *scr):
    NB = 4
    g_v, sq_v, sc_v = scr[0], scr[1], scr[2]
    rows = scr[3:3 + NB]
    const_v = scr[3 + NB]
    sems = scr[4 + NB:4 + 2 * NB]
    acc_c, acc_q = scr[4 + 2 * NB], scr[5 + 2 * NB]
    c = lax.axis_index("c")
    s = lax.axis_index("s")
    w = c * NS + s
    pltpu.sync_copy(gidx.at[w], g_v)
    pltpu.sync_copy(sidxq.at[w], sq_v)
    pltpu.sync_copy(sidxc.at[w], sc_v)
    pltpu.sync_copy(const_rows, const_v)
    pltpu.sync_copy(zrows, acc_c.at[pl.ds(s * STRIPE, STRIPE)])
    pltpu.sync_copy(zrows, acc_q.at[pl.ds(s * STRIPE, STRIPE)])
    plsc.subcore_barrier()

    for b in range(NB - 1):
        pltpu.async_copy(l0.at[g_v.at[b]], rows[b], sems[b])

    def body(j, carry):
        ja = NB * j
        pltpu.async_copy(l0.at[g_v.at[ja + NB - 1]], rows[NB - 1],
                         sems[NB - 1])
        for b in range(NB):
            pltpu.make_async_copy(
                l0.at[g_v.at[ja + b]], rows[b], sems[b]).wait()
            pltpu.sync_copy(rows[b], acc_q.at[sq_v.at[ja + b]], add=True)
            pltpu.sync_copy(const_v, acc_c.at[sc_v.at[ja + b]], add=True)
            if b < NB - 1:
                @pl.when(ja + NB + b < NCH)
                def _():
                    pltpu.async_copy(
                        l0.at[g_v.at[ja + NB + b]], rows[b], sems[b])
        return carry

    lax.fori_loop(0, NCH // 4, body, 0)
    plsc.subcore_barrier()
    pltpu.sync_copy(acc_c.at[pl.ds(s * STRIPE, STRIPE)],
                    cnt_out.at[c].at[pl.ds(s * STRIPE, STRIPE)])
    pltpu.sync_copy(acc_q.at[pl.ds(s * STRIPE, STRIPE)],
                    q_out.at[c].at[pl.ds(s * STRIPE, STRIPE)])


def _sig1():
    return 1.0 / (1.0 + jnp.exp(jnp.float32(-1.0)))


def _onehot_body(yrep, l0_o):
    cidx = lax.broadcasted_iota(jnp.int32, (1, 128), 1) % C
    l0_o[...] = (yrep[...] == cidx).astype(jnp.float32)


def _mm1_body(x, w1, t1_o):
    t1_o[...] = jnp.dot(x[...], w1[...], preferred_element_type=jnp.float32)


def _scale_body(degp, t1, dinv_o, g1_o):
    deg = _sig1() * (degp[0, :N, 0] + degp[1, :N, 0]) + 1.0
    dinv = lax.rsqrt(deg)[:, None]
    dinv_o[...] = dinv
    g1_o[...] = (_sig1() * dinv) * t1[...]


def _layer_body(p, t, dinv_r, b, gm, bt, wn, tn_o, gn_o):
    dinv = dinv_r[...]
    z = dinv * (p[0, :N, :] + p[1, :N, :]) + (dinv * dinv) * t[...] + b[...]
    mu = jnp.mean(z, axis=0, keepdims=True)
    var = jnp.mean((z - mu) ** 2, axis=0, keepdims=True)
    h = jnp.maximum((z - mu) * lax.rsqrt(var + 1e-5) * gm[...] + bt[...], 0.0)
    tn = jnp.dot(h, wn[...], preferred_element_type=jnp.float32)
    tn_o[...] = tn
    gn_o[...] = (_sig1() * dinv) * tn


def _lpa_body(q, lprev, l_o):
    l_o[...] = _sig1() * (q[0, :N // 8, :] + q[1, :N // 8, :]) + lprev[...]


def _final_body(p, t, dinv_r, b, out_o):
    dinv = dinv_r[...]
    z = dinv * (p[0, :N, :] + p[1, :N, :]) + (dinv * dinv) * t[...] + b[...]
    m = jnp.max(z, axis=1, keepdims=True)
    e = jnp.exp(z - m)
    out_o[...] = e / jnp.sum(e, axis=1, keepdims=True)


def _lpa_final_body(q, lprev, l_o):
    l2 = _sig1() * (q[0, :N, :] + q[1, :N, :]) + lprev[...]
    nrm = jnp.sqrt(jnp.sum(l2 * l2, axis=1, keepdims=True))
    l_o[...] = l2 / jnp.maximum(nrm, 1e-12)


def _tc(body, out_shape, *args):
    return pl.pallas_call(body, out_shape=out_shape)(*args)


_f32 = jnp.float32


def kernel(x, edge_index, y, edge_weight, W1, b1, W2, b2, W3, b3,
           g1, bt1, g2, bt2):
    padn = EPAD - E
    NR = N // 8
    NPR = NPAD // 8
    ER = E // K
    PR = padn // K
    ei3 = edge_index.reshape(2, ER, K)
    pidx2 = jnp.arange(padn, dtype=jnp.int32).reshape(PR, K)
    pad_g2 = pidx2 % N
    pad_s2 = N + pidx2 % (NPAD - N)
    gidx_conv = jnp.concatenate([ei3[0], pad_g2], 0).reshape(NW, NCH, K)
    sidx_conv = jnp.concatenate([ei3[1], pad_s2], 0).reshape(NW, NCH, K)
    gidx_lpa = jnp.concatenate([ei3[1], pad_g2], 0).reshape(NW, NCH, K)
    sidx_lpa = jnp.concatenate([ei3[0], pad_s2], 0).reshape(NW, NCH, K)
    const_rows = jnp.tile(
        (jnp.arange(C, dtype=jnp.int32) == 0).astype(_f32)[None, :], (K, 1))
    zrows128 = jnp.zeros((STRIPE, D_H), _f32)
    zrowsC = jnp.zeros((STRIPE, C), _f32)
    yrep = jnp.repeat(y.astype(jnp.int32).reshape(NR, 8), C, axis=1)
    b1r, b2r = b1.reshape(1, D_H), b2.reshape(1, D_H)
    b3r = b3.reshape(1, C)
    g1r, bt1r = g1.reshape(1, D_H), bt1.reshape(1, D_H)
    g2r, bt2r = g2.reshape(1, D_H), bt2.reshape(1, D_H)

    L0_128 = _tc(_onehot_body, jax.ShapeDtypeStruct((NR, 128), _f32), yrep)
    L0 = L0_128.reshape(N, C)
    t1 = _tc(_mm1_body, jax.ShapeDtypeStruct((N, D_H), _f32), x, W1)
    degp, q1 = _cnt_lpa_kernel(L0, gidx_lpa, sidx_lpa, sidx_conv,
                               const_rows, zrowsC)
    dinv, g1t = _tc(
        _scale_body,
        (jax.ShapeDtypeStruct((N, 1), _f32),
         jax.ShapeDtypeStruct((N, D_H), _f32)),
        degp, t1)

    p1 = _agg128(g1t, gidx_conv, sidx_conv, zrows128)
    t2, g2t = _tc(
        _layer_body,
        (jax.ShapeDtypeStruct((N, D_H), _f32),
         jax.ShapeDtypeStruct((N, D_H), _f32)),
        p1, t1, dinv, b1r, g1r, bt1r, W2)

    p2 = _agg128(g2t, gidx_conv, sidx_conv, zrows128)
    t3, g3t = _tc(
        _layer_body,
        (jax.ShapeDtypeStruct((N, C), _f32),
         jax.ShapeDtypeStruct((N, C), _f32)),
        p2, t2, dinv, b2r, g2r, bt2r, W3)
    L1_128 = _tc(_lpa_body, jax.ShapeDtypeStruct((NR, 128), _f32),
                 q1.reshape(NC, NPR, 128), L0_128)
    L1 = L1_128.reshape(N, C)

    p3 = _aggC(g3t, gidx_conv, sidx_conv, zrowsC)
    q2 = _aggC(L1, gidx_lpa, sidx_lpa, zrowsC)
    out1 = _tc(_final_body, jax.ShapeDtypeStruct((N, C), _f32),
               p3, t3, dinv, b3r)
    labels = _tc(_lpa_final_body, jax.ShapeDtypeStruct((N, C), _f32), q2, L1)

    return (out1, labels)


# speedup vs baseline: 1.1221x; 1.0247x over previous
import functools

import jax
import jax.numpy as jnp
from jax import lax
from jax.experimental import pallas as pl
from jax.experimental.pallas import tpu as pltpu
from jax.experimental.pallas import tpu_sc as plsc

N = 10000
E = 160000
D_IN = 128
D_H = 128
C = 16

NC = 2
NS = 16
NW = NC * NS
L = 16

NPAD = 10240
STRIPE = NPAD // NS
K = 128
NCH = 40
EPAD = NW * NCH * K
EW_PER = NCH * K

_mesh = plsc.VectorSubcoreMesh(core_axis_name="c", subcore_axis_name="s")


def _make_agg(D, NB):
    @functools.partial(
        pl.kernel,
        mesh=_mesh,
        out_type=jax.ShapeDtypeStruct((NC, NPAD, D), jnp.float32),
        compiler_params=pltpu.CompilerParams(use_tc_tiling_on_sc=(D >= 128)),
        scratch_types=(
            [pltpu.VMEM((NCH, K), jnp.int32),
             pltpu.VMEM((NCH, K), jnp.int32)]
            + [pltpu.VMEM((K, D), jnp.float32)] * NB
            + [pltpu.SemaphoreType.DMA] * NB
            + [pltpu.VMEM_SHARED((NPAD, D), jnp.float32)]
        ),
    )
    def agg(table, gidx, sidx, zrows, out, *scr):
        gidx_v, sidx_v = scr[0], scr[1]
        rows = scr[2:2 + NB]
        sems = scr[2 + NB:2 + 2 * NB]
        acc = scr[2 + 2 * NB]
        c = lax.axis_index("c")
        s = lax.axis_index("s")
        w = c * NS + s
        pltpu.sync_copy(gidx.at[w], gidx_v)
        pltpu.sync_copy(sidx.at[w], sidx_v)
        pltpu.sync_copy(zrows, acc.at[pl.ds(s * STRIPE, STRIPE)])
        plsc.subcore_barrier()

        for b in range(NB - 1):
            pltpu.async_copy(table.at[gidx_v.at[b]], rows[b], sems[b])

        def body(j, carry):
            ja = NB * j
            pltpu.async_copy(
                table.at[gidx_v.at[ja + NB - 1]], rows[NB - 1], sems[NB - 1])
            for b in range(NB):
                pltpu.make_async_copy(
                    table.at[gidx_v.at[ja + b]], rows[b], sems[b]).wait()
                pltpu.sync_copy(rows[b], acc.at[sidx_v.at[ja + b]], add=True)
                if b < NB - 1:
                    @pl.when(ja + NB + b < NCH)
                    def _():
                        pltpu.async_copy(
                            table.at[gidx_v.at[ja + NB + b]], rows[b], sems[b])
            return carry

        lax.fori_loop(0, NCH // NB, body, 0)
        plsc.subcore_barrier()
        pltpu.sync_copy(acc.at[pl.ds(s * STRIPE, STRIPE)],
                        out.at[c].at[pl.ds(s * STRIPE, STRIPE)])

    return agg


_agg128 = _make_agg(D_H, 2)
_aggC = _make_agg(C, 4)


@functools.partial(
    pl.kernel,
    mesh=_mesh,
    out_type=(jax.ShapeDtypeStruct((NC, NPAD, C), jnp.float32),
              jax.ShapeDtypeStruct((NC, NPAD, C), jnp.float32)),
    compiler_params=pltpu.CompilerParams(use_tc_tiling_on_sc=False),
    scratch_types=(
        [pltpu.VMEM((NCH, K), jnp.int32),
         pltpu.VMEM((NCH, K), jnp.int32),
         pltpu.VMEM((NCH, K), jnp.int32)]
        + [pltpu.VMEM((K, C), jnp.float32)] * 4
        + [pltpu.VMEM((K, C), jnp.float32)]
        + [pltpu.SemaphoreType.DMA] * 4
        + [pltpu.VMEM_SHARED((NPAD, C), jnp.float32),
           pltpu.VMEM_SHARED((NPAD, C), jnp.float32)]
    ),
)
def _cnt_lpa_kernel(l0, gidx, sidxq, sidxc, const_rows, zrows, cnt_out, q_out,
                    *scr):
    NB = 4
    g_v, sq_v, sc_v = scr[0], scr[1], scr[2]
    rows = scr[3:3 + NB]
    const_v = scr[3 + NB]
    sems = scr[4 + NB:4 + 2 * NB]
    acc_c, acc_q = scr[4 + 2 * NB], scr[5 + 2 * NB]
    c = lax.axis_index("c")
    s = lax.axis_index("s")
    w = c * NS + s
    pltpu.sync_copy(gidx.at[w], g_v)
    pltpu.sync_copy(sidxq.at[w], sq_v)
    pltpu.sync_copy(sidxc.at[w], sc_v)
    pltpu.sync_copy(const_rows, const_v)
    pltpu.sync_copy(zrows, acc_c.at[pl.ds(s * STRIPE, STRIPE)])
    pltpu.sync_copy(zrows, acc_q.at[pl.ds(s * STRIPE, STRIPE)])
    plsc.subcore_barrier()

    for b in range(NB - 1):
        pltpu.async_copy(l0.at[g_v.at[b]], rows[b], sems[b])

    def body(j, carry):
        ja = NB * j
        pltpu.async_copy(l0.at[g_v.at[ja + NB - 1]], rows[NB - 1],
                         sems[NB - 1])
        for b in range(NB):
            pltpu.make_async_copy(
                l0.at[g_v.at[ja + b]], rows[b], sems[b]).wait()
            pltpu.sync_copy(rows[b], acc_q.at[sq_v.at[ja + b]], add=True)
            pltpu.sync_copy(const_v, acc_c.at[sc_v.at[ja + b]], add=True)
            if b < NB - 1:
                @pl.when(ja + NB + b < NCH)
                def _():
                    pltpu.async_copy(
                        l0.at[g_v.at[ja + NB + b]], rows[b], sems[b])
        return carry

    lax.fori_loop(0, NCH // 4, body, 0)
    plsc.subcore_barrier()
    pltpu.sync_copy(acc_c.at[pl.ds(s * STRIPE, STRIPE)],
                    cnt_out.at[c].at[pl.ds(s * STRIPE, STRIPE)])
    pltpu.sync_copy(acc_q.at[pl.ds(s * STRIPE, STRIPE)],
                    q_out.at[c].at[pl.ds(s * STRIPE, STRIPE)])


def _sig1():
    return 1.0 / (1.0 + jnp.exp(jnp.float32(-1.0)))


def _onehot_body(yrep, l0_o):
    cidx = lax.broadcasted_iota(jnp.int32, (1, 128), 1) % C
    l0_o[...] = (yrep[...] == cidx).astype(jnp.float32)


def _mm1_body(x, w1, t1_o):
    t1_o[...] = jnp.dot(x[...], w1[...], preferred_element_type=jnp.float32)


def _scale_body(degp, t1, dinv_o, g1_o):
    deg = _sig1() * (degp[0, :N, 0] + degp[1, :N, 0]) + 1.0
    dinv = lax.rsqrt(deg)[:, None]
    dinv_o[...] = dinv
    g1_o[...] = (_sig1() * dinv) * t1[...]


def _layer_body(p, t, dinv_r, b, gm, bt, wn, tn_o, gn_o):
    dinv = dinv_r[...]
    z = dinv * (p[0, :N, :] + p[1, :N, :]) + (dinv * dinv) * t[...] + b[...]
    mu = jnp.mean(z, axis=0, keepdims=True)
    var = jnp.mean((z - mu) ** 2, axis=0, keepdims=True)
    h = jnp.maximum((z - mu) * lax.rsqrt(var + 1e-5) * gm[...] + bt[...], 0.0)
    tn = jnp.dot(h, wn[...], preferred_element_type=jnp.float32)
    tn_o[...] = tn
    gn_o[...] = (_sig1() * dinv) * tn


def _lpa_body(q, lprev, l_o):
    l_o[...] = _sig1() * (q[0, :N // 8, :] + q[1, :N // 8, :]) + lprev[...]


def _final_body(p, t, dinv_r, b, out_o):
    dinv = dinv_r[...]
    z = dinv * (p[0, :N, :] + p[1, :N, :]) + (dinv * dinv) * t[...] + b[...]
    m = jnp.max(z, axis=1, keepdims=True)
    e = jnp.exp(z - m)
    out_o[...] = e / jnp.sum(e, axis=1, keepdims=True)


def _lpa_final_body(q, lprev, l_o):
    l2 = _sig1() * (q[0, :N // 8, :] + q[1, :N // 8, :]) + lprev[...]
    ki = lax.broadcasted_iota(jnp.int32, (128, 8), 0) // C
    ai = lax.broadcasted_iota(jnp.int32, (128, 8), 1)
    m = (ki == ai).astype(jnp.float32)
    ssq = jnp.dot(l2 * l2, m, preferred_element_type=jnp.float32)
    rinv = 1.0 / jnp.maximum(jnp.sqrt(ssq), 1e-12)
    rexp = jnp.dot(rinv, m.T, preferred_element_type=jnp.float32)
    l_o[...] = l2 * rexp


def _tc(body, out_shape, *args):
    return pl.pallas_call(body, out_shape=out_shape)(*args)


_f32 = jnp.float32


def kernel(x, edge_index, y, edge_weight, W1, b1, W2, b2, W3, b3,
           g1, bt1, g2, bt2):
    padn = EPAD - E
    NR = N // 8
    NPR = NPAD // 8
    ER = E // K
    PR = padn // K
    ei3 = edge_index.reshape(2, ER, K)
    pidx2 = jnp.arange(padn, dtype=jnp.int32).reshape(PR, K)
    pad_g2 = pidx2 % N
    pad_s2 = N + pidx2 % (NPAD - N)
    gidx_conv = jnp.concatenate([ei3[0], pad_g2], 0).reshape(NW, NCH, K)
    sidx_conv = jnp.concatenate([ei3[1], pad_s2], 0).reshape(NW, NCH, K)
    gidx_lpa = jnp.concatenate([ei3[1], pad_g2], 0).reshape(NW, NCH, K)
    sidx_lpa = jnp.concatenate([ei3[0], pad_s2], 0).reshape(NW, NCH, K)
    const_rows = jnp.tile(
        (jnp.arange(C, dtype=jnp.int32) == 0).astype(_f32)[None, :], (K, 1))
    zrows128 = jnp.zeros((STRIPE, D_H), _f32)
    zrowsC = jnp.zeros((STRIPE, C), _f32)
    yrep = jnp.repeat(y.astype(jnp.int32).reshape(NR, 8), C, axis=1)
    b1r, b2r = b1.reshape(1, D_H), b2.reshape(1, D_H)
    b3r = b3.reshape(1, C)
    g1r, bt1r = g1.reshape(1, D_H), bt1.reshape(1, D_H)
    g2r, bt2r = g2.reshape(1, D_H), bt2.reshape(1, D_H)

    L0_128 = _tc(_onehot_body, jax.ShapeDtypeStruct((NR, 128), _f32), yrep)
    L0 = L0_128.reshape(N, C)
    t1 = _tc(_mm1_body, jax.ShapeDtypeStruct((N, D_H), _f32), x, W1)
    degp, q1 = _cnt_lpa_kernel(L0, gidx_lpa, sidx_lpa, sidx_conv,
                               const_rows, zrowsC)
    dinv, g1t = _tc(
        _scale_body,
        (jax.ShapeDtypeStruct((N, 1), _f32),
         jax.ShapeDtypeStruct((N, D_H), _f32)),
        degp, t1)

    p1 = _agg128(g1t, gidx_conv, sidx_conv, zrows128)
    t2, g2t = _tc(
        _layer_body,
        (jax.ShapeDtypeStruct((N, D_H), _f32),
         jax.ShapeDtypeStruct((N, D_H), _f32)),
        p1, t1, dinv, b1r, g1r, bt1r, W2)

    p2 = _agg128(g2t, gidx_conv, sidx_conv, zrows128)
    t3, g3t = _tc(
        _layer_body,
        (jax.ShapeDtypeStruct((N, C), _f32),
         jax.ShapeDtypeStruct((N, C), _f32)),
        p2, t2, dinv, b2r, g2r, bt2r, W3)
    L1_128 = _tc(_lpa_body, jax.ShapeDtypeStruct((NR, 128), _f32),
                 q1.reshape(NC, NPR, 128), L0_128)
    L1 = L1_128.reshape(N, C)

    p3 = _aggC(g3t, gidx_conv, sidx_conv, zrowsC)
    q2 = _aggC(L1, gidx_lpa, sidx_lpa, zrowsC)
    out1 = _tc(_final_body, jax.ShapeDtypeStruct((N, C), _f32),
               p3, t3, dinv, b3r)
    labels128 = _tc(_lpa_final_body, jax.ShapeDtypeStruct((NR, 128), _f32),
                    q2.reshape(NC, NPR, 128), L1_128)
    labels = labels128.reshape(N, C)

    return (out1, labels)
